# Initial kernel scaffold; baseline (speedup 1.0000x reference)
import jax
import jax.numpy as jnp
from jax.experimental import pallas as pl


def kernel(x, weight):
    raise NotImplementedError("write your pallas kernel here")


# speedup vs baseline: 14.6146x; 14.6146x over previous
import functools

import jax
import jax.numpy as jnp
import numpy as np
from jax.experimental import pallas as pl

N_WIDTH = 32
N_ORDER = 5
N_ELEMENTS = 16
N_NODES = N_ELEMENTS * N_ORDER + 1
X_MIN = 0.0
X_MAX = 1.0

_NODES = np.linspace(-1.0, 1.0, N_ORDER + 1)


def _basis_all(x):
    nd = _NODES
    phi, dphi, ddphi = [], [], []
    for j in range(N_ORDER + 1):
        p = None
        for m in range(N_ORDER + 1):
            if m != j:
                f = (x - nd[m]) * (1.0 / (nd[j] - nd[m]))
                p = f if p is None else p * f
        phi.append(p)

        y = None
        for i in range(N_ORDER + 1):
            if i == j:
                continue
            k = None
            for m in range(N_ORDER + 1):
                if m != i and m != j:
                    f = (x - nd[m]) * (1.0 / (nd[j] - nd[m]))
                    k = f if k is None else k * f
            k = k * (1.0 / (nd[j] - nd[i]))
            y = k if y is None else y + k
        dphi.append(y)

        y2 = None
        for i in range(N_ORDER + 1):
            if i == j:
                continue
            ks = None
            for m in range(N_ORDER + 1):
                if m != i and m != j:
                    kp = None
                    for n in range(N_ORDER + 1):
                        if n != i and n != j and n != m:
                            f = (x - nd[n]) * (1.0 / (nd[j] - nd[n]))
                            kp = f if kp is None else kp * f
                    kp = kp * (1.0 / (nd[j] - nd[m]))
                    ks = kp if ks is None else ks + kp
            ks = ks * (1.0 / (nd[j] - nd[i]))
            y2 = ks if y2 is None else y2 + ks
        ddphi.append(y2)
    return phi, dphi, ddphi


def _block_kernel(x_ref, w_ref, t_ref, dt_ref, ddt_ref,
                  phi_ref, dphi_ref, ddphi_ref, *, bs):
    x = x_ref[...]
    x_shift = (N_NODES - 1) * (x - X_MIN) * (1.0 / (X_MAX - X_MIN))
    id_el = jnp.clip(jnp.floor(x_shift * (1.0 / N_ORDER)), 0.0, N_ELEMENTS - 1)
    nodes_l = id_el * N_ORDER
    x_t = 2.0 * (x_shift - nodes_l) * (1.0 / N_ORDER) - 1.0

    delta_x = 0.5 * N_ORDER * (X_MAX - X_MIN) / (N_NODES - 1)
    inv_dx = 1.0 / delta_x
    phi_l, dphi_l, ddphi_l = _basis_all(x_t)
    dphi_l = [v * inv_dx for v in dphi_l]
    ddphi_l = [v * (inv_dx * inv_dx) for v in ddphi_l]

    p_iota = jax.lax.broadcasted_iota(jnp.int32, (bs, N_WIDTH, N_NODES), 2)
    rel = p_iota - nodes_l.astype(jnp.int32)[..., None]

    w = w_ref[...][None, :, :]

    for locs, out_ref, red_ref in (
        (phi_l, phi_ref, t_ref),
        (dphi_l, dphi_ref, dt_ref),
        (ddphi_l, ddphi_ref, ddt_ref),
    ):
        acc = jnp.zeros((bs, N_WIDTH, N_NODES), jnp.float32)
        for j in range(N_ORDER + 1):
            acc = acc + jnp.where(rel == j, locs[j][..., None], 0.0)
        out_ref[...] = acc
        red_ref[...] = jnp.sum(acc * w, axis=-1)


@jax.jit
def kernel(x, weight):
    if x.ndim != 2:
        x = jnp.repeat(x[..., None], N_WIDTH, axis=-1)
    S, W = x.shape
    bs = 256
    while S % bs != 0:
        bs //= 2
    grid = (S // bs,)

    out_shapes = (
        jax.ShapeDtypeStruct((S, W), jnp.float32),
        jax.ShapeDtypeStruct((S, W), jnp.float32),
        jax.ShapeDtypeStruct((S, W), jnp.float32),
        jax.ShapeDtypeStruct((S, W, N_NODES), jnp.float32),
        jax.ShapeDtypeStruct((S, W, N_NODES), jnp.float32),
        jax.ShapeDtypeStruct((S, W, N_NODES), jnp.float32),
    )
    spec2 = pl.BlockSpec((bs, W), lambda i: (i, 0))
    spec3 = pl.BlockSpec((bs, W, N_NODES), lambda i: (i, 0, 0))
    out = pl.pallas_call(
        functools.partial(_block_kernel, bs=bs),
        grid=grid,
        in_specs=[spec2, pl.BlockSpec((W, N_NODES), lambda i: (0, 0))],
        out_specs=(spec2, spec2, spec2, spec3, spec3, spec3),
        out_shape=out_shapes,
    )(x, weight)
    return out
